# baseline (speedup 1.0000x reference)
import jax
import jax.numpy as jnp
from jax.experimental import pallas as pl
from jax.experimental.pallas import tpu as pltpu

_NUM_EMB = 1024
_DIM = 64
_HW = 1024
_IMGS = 16
_ROWS = _IMGS * _HW
_COMMITMENT = 0.25
_NBUF = 4
_CHUNK = 64
_NCHUNK = _NUM_EMB // _CHUNK


def _enc_copy(scratch_ref, enc_ref, sem, buf, img):
    return pltpu.make_async_copy(
        scratch_ref.at[buf],
        enc_ref.at[pl.ds(img * _HW, _HW), :],
        sem.at[buf],
    )


def _vq_body(x_ref, xsq_ref, emb_ref, embt_ref, esq_ref,
             enc_ref, q_ref, loss_ref, scratch_ref, oh_ref, sem):
    step = pl.program_id(0)
    buf = step % _NBUF
    x = x_ref[0]
    mt = jax.lax.dot_general(emb_ref[...], x,
                             (((1,), (0,)), ((), ())),
                             preferred_element_type=jnp.float32)
    xsq = xsq_ref[0]
    iota_c = jax.lax.broadcasted_iota(
        jnp.int32, (_CHUNK, _HW), 0).astype(jnp.float32)

    run_min = None
    run_idx = None
    for c in range(_NCHUNK):
        lo, hi = c * _CHUNK, (c + 1) * _CHUNK
        dtc = (xsq + esq_ref[lo:hi, :]) - 2.0 * mt[lo:hi, :]
        cmin = jnp.min(dtc, axis=0, keepdims=True)
        cidx = jnp.min(jnp.where(dtc == cmin, iota_c, float(_CHUNK)),
                       axis=0, keepdims=True) + float(c * _CHUNK)
        if c == 0:
            run_min, run_idx = cmin, cidx
        else:
            upd = cmin < run_min
            run_min = jnp.where(upd, cmin, run_min)
            run_idx = jnp.where(upd, cidx, run_idx)

    for c in range(_NCHUNK):
        oh_ref[c * _CHUNK:(c + 1) * _CHUNK, :] = jnp.where(
            iota_c == run_idx - float(c * _CHUNK), 1.0, 0.0)

    @pl.when(step >= _NBUF)
    def _():
        _enc_copy(scratch_ref, enc_ref, sem, buf, step - _NBUF).wait()

    onehot_t = oh_ref[...]
    scratch_ref[buf] = onehot_t.T
    _enc_copy(scratch_ref, enc_ref, sem, buf, step).start()

    q = jnp.dot(embt_ref[...], onehot_t,
                preferred_element_type=jnp.float32)
    q_ref[0] = x + (q - x)

    @pl.when(step == 0)
    def _():
        loss_ref[...] = jnp.zeros_like(loss_ref)

    loss_ref[...] += jnp.sum(run_min).reshape(1, 1)

    @pl.when(step == _IMGS - 1)
    def _():
        for off in range(_NBUF):
            img = _IMGS - 1 - off
            _enc_copy(scratch_ref, enc_ref, sem, img % _NBUF, img).wait()


def kernel(inputs, embedding):
    x_chw = inputs.astype(jnp.float32).reshape(_IMGS, _DIM, _HW)
    emb = embedding.astype(jnp.float32)
    flat = jnp.transpose(inputs, (0, 2, 3, 1)).reshape(-1, _DIM)
    flat = flat.astype(jnp.float32)
    xsq = jnp.sum(flat ** 2, axis=1).reshape(_IMGS, 1, _HW)
    esq = jnp.sum(emb ** 2, axis=1)[:, None]
    embt = emb.T

    enc, q, loss_sum = pl.pallas_call(
        _vq_body,
        grid=(_IMGS,),
        in_specs=[
            pl.BlockSpec((1, _DIM, _HW), lambda i: (i, 0, 0)),
            pl.BlockSpec((1, 1, _HW), lambda i: (i, 0, 0)),
            pl.BlockSpec((_NUM_EMB, _DIM), lambda i: (0, 0)),
            pl.BlockSpec((_DIM, _NUM_EMB), lambda i: (0, 0)),
            pl.BlockSpec((_NUM_EMB, 1), lambda i: (0, 0)),
        ],
        out_specs=[
            pl.BlockSpec(memory_space=pl.ANY),
            pl.BlockSpec((1, _DIM, _HW), lambda i: (i, 0, 0)),
            pl.BlockSpec((1, 1), lambda i: (0, 0)),
        ],
        out_shape=[
            jax.ShapeDtypeStruct((_ROWS, _NUM_EMB), jnp.float32),
            jax.ShapeDtypeStruct((_IMGS, _DIM, _HW), jnp.float32),
            jax.ShapeDtypeStruct((1, 1), jnp.float32),
        ],
        scratch_shapes=[
            pltpu.VMEM((_NBUF, _HW, _NUM_EMB), jnp.float32),
            pltpu.VMEM((_NUM_EMB, _HW), jnp.float32),
            pltpu.SemaphoreType.DMA((_NBUF,)),
        ],
    )(x_chw, xsq, emb, embt, esq)

    quantized = q.reshape(inputs.shape)
    loss = _COMMITMENT * (loss_sum[0, 0] / (_ROWS * _DIM))
    return (quantized, loss, enc)


# speedup vs baseline: 1.1703x; 1.1703x over previous
import jax
import jax.numpy as jnp
from jax.experimental import pallas as pl

_NUM_EMB = 1024
_DIM = 64
_HW = 1024
_IMGS = 16
_ROWS = _IMGS * _HW
_COMMITMENT = 0.25


def _vq_body(x_ref, xsq_ref, emb_ref, embt_ref, esq_ref,
             enc_ref, q_ref, loss_ref):
    step = pl.program_id(0)
    x = x_ref[0]
    mt = jax.lax.dot_general(emb_ref[...], x,
                             (((1,), (0,)), ((), ())),
                             preferred_element_type=jnp.float32)
    dt = (xsq_ref[0] + esq_ref[...]) - 2.0 * mt
    dmin = jnp.min(dt, axis=0, keepdims=True)
    iota = jax.lax.broadcasted_iota(jnp.int32, dt.shape, 0).astype(jnp.float32)
    cand = jnp.where(dt == dmin, iota, float(_NUM_EMB))
    idx = jnp.min(cand, axis=0, keepdims=True)
    onehot_t = jnp.where(cand == idx, 1.0, 0.0)
    enc_ref[...] = onehot_t.T
    q = jnp.dot(embt_ref[...], onehot_t,
                preferred_element_type=jnp.float32)
    q_ref[0] = x + (q - x)

    @pl.when(step == 0)
    def _():
        loss_ref[...] = jnp.zeros_like(loss_ref)

    loss_ref[...] += jnp.sum(dmin).reshape(1, 1)


def kernel(inputs, embedding):
    x_chw = inputs.astype(jnp.float32).reshape(_IMGS, _DIM, _HW)
    emb = embedding.astype(jnp.float32)
    flat = jnp.transpose(inputs, (0, 2, 3, 1)).reshape(-1, _DIM)
    flat = flat.astype(jnp.float32)
    xsq = jnp.sum(flat ** 2, axis=1).reshape(_IMGS, 1, _HW)
    esq = jnp.sum(emb ** 2, axis=1)[:, None]
    embt = emb.T

    enc, q, loss_sum = pl.pallas_call(
        _vq_body,
        grid=(_IMGS,),
        in_specs=[
            pl.BlockSpec((1, _DIM, _HW), lambda i: (i, 0, 0)),
            pl.BlockSpec((1, 1, _HW), lambda i: (i, 0, 0)),
            pl.BlockSpec((_NUM_EMB, _DIM), lambda i: (0, 0)),
            pl.BlockSpec((_DIM, _NUM_EMB), lambda i: (0, 0)),
            pl.BlockSpec((_NUM_EMB, 1), lambda i: (0, 0)),
        ],
        out_specs=[
            pl.BlockSpec((_HW, _NUM_EMB), lambda i: (i, 0)),
            pl.BlockSpec((1, _DIM, _HW), lambda i: (i, 0, 0)),
            pl.BlockSpec((1, 1), lambda i: (0, 0)),
        ],
        out_shape=[
            jax.ShapeDtypeStruct((_ROWS, _NUM_EMB), jnp.float32),
            jax.ShapeDtypeStruct((_IMGS, _DIM, _HW), jnp.float32),
            jax.ShapeDtypeStruct((1, 1), jnp.float32),
        ],
    )(x_chw, xsq, emb, embt, esq)

    quantized = q.reshape(inputs.shape)
    loss = _COMMITMENT * (loss_sum[0, 0] / (_ROWS * _DIM))
    return (quantized, loss, enc)
